# baseline (speedup 1.0000x reference)
import math

import jax
import jax.numpy as jnp
from jax import lax
from jax.experimental import pallas as pl
from jax.experimental.pallas import tpu as pltpu
from jax.experimental.pallas import tpu_sc as plsc

DIM = 64
LANES = 16
NC, NS = 2, 16
NW = NC * NS
CHUNK = 512
NVEC = CHUNK // LANES
SCALE = math.sqrt(DIM)


def _gather_start(table_hbm, idx_v, c, rows, gsem):
    def g(k, carry):
        vec = idx_v[pl.ds(c * CHUNK + k * LANES, LANES)]
        pltpu.async_copy(
            table_hbm.at[vec], rows.at[pl.ds(k * LANES, LANES)], gsem
        )
        return carry

    lax.fori_loop(0, NVEC, g, jnp.int32(0))


def _gather_wait(table_hbm, rows, gsem):
    pltpu.make_async_copy(table_hbm.at[pl.ds(0, CHUNK)], rows, gsem).wait()


def _scale(rows):
    @plsc.parallel_loop(0, CHUNK, step=1)
    def _body(i):
        for j in range(DIM // LANES):
            rows[i, pl.ds(j * LANES, LANES)] = (
                rows[i, pl.ds(j * LANES, LANES)] * SCALE
            )


def _emb_body(x_hbm, table_hbm, out_hbm, idx_v, rows0, rows1,
              gsem0, gsem1, ssem0, ssem1):
    wid = lax.axis_index("s") * NC + lax.axis_index("c")
    nchunk = out_hbm.shape[1]
    nloop = nchunk // 2

    pltpu.sync_copy(x_hbm.at[wid], idx_v)
    _gather_start(table_hbm, idx_v, 0, rows0, gsem0)

    def pair(i, carry):
        c0 = 2 * i

        @pl.when(i > 0)
        def _():
            pltpu.make_async_copy(rows1, out_hbm.at[wid, c0], ssem1).wait()

        _gather_start(table_hbm, idx_v, c0 + 1, rows1, gsem1)
        _gather_wait(table_hbm, rows0, gsem0)
        _scale(rows0)
        pltpu.async_copy(rows0, out_hbm.at[wid, c0], ssem0)

        @pl.when(i < nloop - 1)
        def _():
            pltpu.make_async_copy(rows0, out_hbm.at[wid, c0], ssem0).wait()
            _gather_start(table_hbm, idx_v, c0 + 2, rows0, gsem0)

        _gather_wait(table_hbm, rows1, gsem1)
        _scale(rows1)
        pltpu.async_copy(rows1, out_hbm.at[wid, c0 + 1], ssem1)
        return carry

    lax.fori_loop(0, nloop, pair, jnp.int32(0))
    pltpu.make_async_copy(rows0, out_hbm.at[wid, nchunk - 2], ssem0).wait()
    pltpu.make_async_copy(rows1, out_hbm.at[wid, nchunk - 1], ssem1).wait()


@jax.jit
def kernel(x, table):
    b, l = x.shape
    total = b * l
    nchunk = total // (NW * CHUNK)
    xr = x.astype(jnp.int32).reshape(NW, nchunk * CHUNK)
    mesh = plsc.VectorSubcoreMesh(
        core_axis_name="c", subcore_axis_name="s",
        num_cores=NC, num_subcores=NS,
    )
    out = pl.kernel(
        _emb_body,
        out_type=jax.ShapeDtypeStruct((NW, nchunk, CHUNK, DIM), jnp.float32),
        mesh=mesh,
        compiler_params=pltpu.CompilerParams(use_tc_tiling_on_sc=False),
        scratch_types=[
            pltpu.VMEM((nchunk * CHUNK,), jnp.int32),
            pltpu.VMEM((CHUNK, DIM), jnp.float32),
            pltpu.VMEM((CHUNK, DIM), jnp.float32),
            pltpu.SemaphoreType.DMA,
            pltpu.SemaphoreType.DMA,
            pltpu.SemaphoreType.DMA,
            pltpu.SemaphoreType.DMA,
        ],
    )(xr, table)
    return out.reshape(b, l, DIM)
